# Initial kernel scaffold; baseline (speedup 1.0000x reference)
import jax
import jax.numpy as jnp
from jax.experimental import pallas as pl


def kernel(x, encoding_weight):
    raise NotImplementedError("write your pallas kernel here")


# speedup vs baseline: 1.9649x; 1.9649x over previous
import jax
import jax.numpy as jnp
from jax.experimental import pallas as pl


def _add_kernel(x_ref, w_ref, o_ref):
    o_ref[...] = x_ref[...] + w_ref[...][None, :, :]


def kernel(x, encoding_weight):
    batch, seq_len, d_model = x.shape
    sblk = 512
    grid = (seq_len // sblk,)
    return pl.pallas_call(
        _add_kernel,
        grid=grid,
        in_specs=[
            pl.BlockSpec((batch, sblk, d_model), lambda i: (0, i, 0)),
            pl.BlockSpec((sblk, d_model), lambda i: (i, 0)),
        ],
        out_specs=pl.BlockSpec((batch, sblk, d_model), lambda i: (0, i, 0)),
        out_shape=jax.ShapeDtypeStruct((batch, seq_len, d_model), x.dtype),
    )(x, encoding_weight)
